# baseline (speedup 1.0000x reference)
import functools

import jax
import jax.numpy as jnp
from jax import lax
from jax.experimental import pallas as pl
from jax.experimental.pallas import tpu as pltpu
from jax.experimental.pallas import tpu_sc as plsc

_B = 16384
_D = 32
_EPS = 1e-5
_CHUNK = 128


def _rsqrt16(x):
    i = plsc.bitcast(x, jnp.int32)
    i = jnp.int32(0x5F3759DF) - lax.shift_right_logical(i, 1)
    y = plsc.bitcast(i, jnp.float32)
    for _ in range(3):
        y = y * (1.5 - 0.5 * x * y * y)
    return y


def _sc_embed_ln(table, idx, b, gamma, beta):
    info = plsc.get_sparse_core_info()
    nc, ns = info.num_cores, info.num_subcores
    nw = nc * ns
    bpw = _B // nw
    nchunk = bpw // _CHUNK
    blk_per_chunk = _CHUNK // 16
    mesh = plsc.VectorSubcoreMesh(core_axis_name="c", subcore_axis_name="s")

    @functools.partial(
        pl.kernel,
        mesh=mesh,
        out_type=jax.ShapeDtypeStruct((_B, _D), jnp.float32),
        scratch_types=[
            pltpu.VMEM((nchunk, _CHUNK), jnp.int32),
            pltpu.VMEM((bpw, _D), jnp.float32),
            pltpu.VMEM((bpw, _D), jnp.float32),
            pltpu.VMEM((_D,), jnp.float32),
            pltpu.VMEM((_D,), jnp.float32),
            pltpu.VMEM((_D,), jnp.float32),
            pltpu.SemaphoreType.DMA,
            pltpu.SemaphoreType.DMA,
        ],
        compiler_params=pltpu.CompilerParams(
            needs_layout_passes=False, use_tc_tiling_on_sc=False),
    )
    def k(table_h, idx_h, b_h, g_h, be_h, out_h,
          idx_v, rows_v, out_v, b_v, g_v, be_v, gsem, wsem):
        wid = lax.axis_index("s") * nc + lax.axis_index("c")
        base = wid * bpw
        for j in range(nchunk):
            pltpu.sync_copy(idx_h.at[pl.ds(base + j * _CHUNK, _CHUNK)],
                            idx_v.at[j])
        pltpu.sync_copy(b_h, b_v)
        pltpu.sync_copy(g_h, g_v)
        pltpu.sync_copy(be_h, be_v)
        gathers = [
            pltpu.async_copy(table_h.at[idx_v.at[j]],
                             rows_v.at[pl.ds(j * _CHUNK, _CHUNK)], gsem)
            for j in range(nchunk)
        ]

        b_lanes = [b_v[pl.ds(0, 16)], b_v[pl.ds(16, 16)]]
        g_lanes = [g_v[pl.ds(0, 16)], g_v[pl.ds(16, 16)]]
        be_lanes = [be_v[pl.ds(0, 16)], be_v[pl.ds(16, 16)]]
        lane = lax.iota(jnp.int32, 16)
        cids = [jnp.full((16,), d, jnp.int32) for d in range(_D)]

        def block(blk, carry):
            rid = blk * 16 + lane
            s = [jnp.zeros((16,), jnp.float32) for _ in range(4)]
            ss = [jnp.zeros((16,), jnp.float32) for _ in range(4)]
            for d in range(_D):
                x = plsc.load_gather(rows_v, [rid, cids[d]]) \
                    + b_lanes[d // 16][d % 16]
                s[d % 4] = s[d % 4] + x
                ss[d % 4] = ss[d % 4] + x * x
            st = (s[0] + s[1]) + (s[2] + s[3])
            sst = (ss[0] + ss[1]) + (ss[2] + ss[3])
            mean = st * (1.0 / _D)
            var = sst * (1.0 / _D) - mean * mean
            r = _rsqrt16(var + _EPS)
            for d in range(_D):
                m_d = mean - b_lanes[d // 16][d % 16]
                rg_d = r * g_lanes[d // 16][d % 16]
                x = plsc.load_gather(rows_v, [rid, cids[d]])
                o = (x - m_d) * rg_d + be_lanes[d // 16][d % 16]
                plsc.store_scatter(out_v, [rid, cids[d]], o)
            return carry

        writes = []
        for j in range(nchunk):
            gathers[j].wait()
            lax.fori_loop(j * blk_per_chunk, (j + 1) * blk_per_chunk,
                          block, 0)
            writes.append(
                pltpu.async_copy(out_v.at[pl.ds(j * _CHUNK, _CHUNK)],
                                 out_h.at[pl.ds(base + j * _CHUNK, _CHUNK)],
                                 wsem))
        for w in writes:
            w.wait()

    return k(table, idx, b, gamma, beta)


def kernel(species_idx, W, b, gamma, beta):
    table = W.T
    idx = species_idx.astype(jnp.int32)
    return _sc_embed_ln(table, idx, b, gamma, beta)
